# baseline (speedup 1.0000x reference)
import functools

import jax
import jax.numpy as jnp
from jax import lax
from jax.experimental import pallas as pl
from jax.experimental.pallas import tpu as pltpu
from jax.experimental.pallas import tpu_sc as plsc

B = 16384
D = 64
NENT = 1000000
NREL = 1000
EPR = 4
PW = 128
NC = 2
NS = 16
L = 16
NW = NC * NS
BPW = B // NW
IDXW = 128
NIDX = BPW // IDXW
CH = 32
NCH = BPW // CH
QW = D // (2 * L)


def _score_body(h1, t1, r1, ent, rel, out, oid_h, oid_t, oid_r,
                tid_h, tid_t, relv, gh0, gh1, gt0, gt1, outv,
                semh0, semh1, semt0, semt1):
    wid = lax.axis_index("s") * NC + lax.axis_index("c")
    base = wid * BPW
    for b in range(NIDX):
        pltpu.sync_copy(h1.at[pl.ds(base + b * IDXW, IDXW)], oid_h.at[b])
        pltpu.sync_copy(t1.at[pl.ds(base + b * IDXW, IDXW)], oid_t.at[b])
        pltpu.sync_copy(r1.at[pl.ds(base + b * IDXW, IDXW)], oid_r.at[b])
    pltpu.sync_copy(rel, relv)
    for b in range(NIDX):
        for u in range(IDXW // L):
            sl = pl.ds(u * L, L)
            tid_h[b, sl] = lax.shift_right_logical(oid_h[b, sl], 2)
            tid_t[b, sl] = lax.shift_right_logical(oid_t[b, sl], 2)
    lanes = lax.iota(jnp.int32, L)

    gh = (gh0, gh1)
    gt = (gt0, gt1)
    semh = (semh0, semh1)
    semt = (semt0, semt1)

    def extract(v, j):
        return jnp.sum(jnp.where(lanes == j, v, 0))

    def issue(ci, s):
        b = ci // (IDXW // CH)
        off = (ci % (IDXW // CH)) * CH
        pltpu.async_copy(
            ent.at[tid_h.at[b, pl.ds(off, CH)]], gh[s], semh[s])
        pltpu.async_copy(
            ent.at[tid_t.at[b, pl.ds(off, CH)]], gt[s], semt[s])

    def drain(s):
        pltpu.make_async_copy(ent.at[pl.ds(0, CH)], gh[s], semh[s]).wait()
        pltpu.make_async_copy(ent.at[pl.ds(0, CH)], gt[s], semt[s]).wait()

    def unpack_row(ref, row, o, q):
        w = ref[row, pl.ds(o + q * L, L)]
        bh = plsc.bitcast(w, jnp.bfloat16)
        return plsc.unpack(bh, format=plsc.PackFormat.INTERLEAVED)

    def compute(ci, s):
        for u in range(CH // L):
            bi = ci * (CH // L) + u
            b2 = bi // (IDXW // L)
            off2 = (bi % (IDXW // L)) * L
            vh = oid_h[b2, pl.ds(off2, L)]
            vt = oid_t[b2, pl.ds(off2, L)]
            vr = oid_r[b2, pl.ds(off2, L)]
            sums = jnp.zeros((L,), jnp.float32)
            for j in range(L):
                hj = extract(vh, j)
                tj = extract(vt, j)
                rj = extract(vr, j)
                oh = jnp.bitwise_and(hj, EPR - 1) * (2 * L)
                ot = jnp.bitwise_and(tj, EPR - 1) * (2 * L)
                mr = lax.shift_right_logical(rj, 2)
                orr = jnp.bitwise_and(rj, EPR - 1) * (2 * L)
                row = u * L + j
                acc = None
                for q in range(QW):
                    a0, a1 = unpack_row(gh[s], row, oh, q)
                    b0, b1 = unpack_row(gt[s], row, ot, q)
                    c0, c1 = unpack_row(relv, mr, orr, q)
                    d0 = (a0 - b0) + c0
                    d1 = (a1 - b1) + c1
                    dd = d0 * d0 + d1 * d1
                    acc = dd if acc is None else acc + dd
                tot = jnp.sum(acc)
                sums = jnp.where(lanes == j, tot, sums)
            x = sums + 1e-12
            ib = plsc.bitcast(x, jnp.int32)
            ib = 0x5F3759DF - lax.shift_right_logical(ib, 1)
            y = plsc.bitcast(ib, jnp.float32)
            for _ in range(3):
                y = y * (1.5 - 0.5 * x * y * y)
            outv[pl.ds(bi * L, L)] = -(x * y)

    issue(0, 0)
    issue(1, 1)

    def body(i, carry):
        drain(0)
        compute(2 * i, 0)

        @pl.when(2 * i + 2 < NCH)
        def _():
            issue(2 * i + 2, 0)

        drain(1)
        compute(2 * i + 1, 1)

        @pl.when(2 * i + 3 < NCH)
        def _():
            issue(2 * i + 3, 1)

        return carry

    lax.fori_loop(0, NCH // 2, body, 0)
    pltpu.sync_copy(outv, out.at[pl.ds(base, BPW)])


@jax.jit
def kernel(h, t, r, entity_emb, relation_emb):
    h1 = h.astype(jnp.int32)
    t1 = t.astype(jnp.int32)
    r1 = r.astype(jnp.int32)
    eb = entity_emb.astype(jnp.bfloat16).reshape(NENT // EPR, PW, 2)
    ei = lax.bitcast_convert_type(eb, jnp.int32)
    rb = relation_emb.astype(jnp.bfloat16).reshape(NREL // EPR, PW, 2)
    ri = lax.bitcast_convert_type(rb, jnp.int32)
    mesh = plsc.VectorSubcoreMesh(
        core_axis_name="c", subcore_axis_name="s",
        num_cores=NC, num_subcores=NS)
    run = pl.kernel(
        _score_body,
        out_type=jax.ShapeDtypeStruct((B,), jnp.float32),
        mesh=mesh,
        compiler_params=pltpu.CompilerParams(needs_layout_passes=False),
        scratch_types=[
            pltpu.VMEM((NIDX, IDXW), jnp.int32),
            pltpu.VMEM((NIDX, IDXW), jnp.int32),
            pltpu.VMEM((NIDX, IDXW), jnp.int32),
            pltpu.VMEM((NIDX, IDXW), jnp.int32),
            pltpu.VMEM((NIDX, IDXW), jnp.int32),
            pltpu.VMEM((NREL // EPR, PW), jnp.int32),
            pltpu.VMEM((CH, PW), jnp.int32),
            pltpu.VMEM((CH, PW), jnp.int32),
            pltpu.VMEM((CH, PW), jnp.int32),
            pltpu.VMEM((CH, PW), jnp.int32),
            pltpu.VMEM((BPW,), jnp.float32),
            pltpu.SemaphoreType.DMA,
            pltpu.SemaphoreType.DMA,
            pltpu.SemaphoreType.DMA,
            pltpu.SemaphoreType.DMA,
        ],
    )
    return run(h1, t1, r1, ei, ri)


# speedup vs baseline: 34.8422x; 34.8422x over previous
import functools

import jax
import jax.numpy as jnp
from jax import lax
from jax.experimental import pallas as pl
from jax.experimental.pallas import tpu as pltpu
from jax.experimental.pallas import tpu_sc as plsc

B = 16384
D = 64
NC = 2
NS = 16
L = 16
NW = NC * NS
BPW = B // NW
CHUNK = 128
NCHUNK = BPW // CHUNK
GROUPS = CHUNK // L


def _score_body(h2, t2, r2, enth, entt, rel, out, idx_h, idx_t, idx_r,
                hrows, trows, rrows, outv, semh, semt, semr):
    wid = lax.axis_index("s") * NC + lax.axis_index("c")
    base_row = wid * NCHUNK
    pltpu.sync_copy(h2.at[pl.ds(base_row, NCHUNK)], idx_h)
    pltpu.sync_copy(t2.at[pl.ds(base_row, NCHUNK)], idx_t)
    pltpu.sync_copy(r2.at[pl.ds(base_row, NCHUNK)], idx_r)
    lanes = lax.iota(jnp.int32, L)

    for c in range(NCHUNK):
        ch = pltpu.async_copy(enth.at[idx_h.at[c]], hrows, semh)
        ct = pltpu.async_copy(entt.at[idx_t.at[c]], trows, semt)
        cr = pltpu.async_copy(rel.at[idx_r.at[c]], rrows, semr)
        ch.wait()
        ct.wait()
        cr.wait()

        def group(g, carry, c=c):
            sums = jnp.zeros((L,), jnp.float32)
            for j in range(L):
                row = g * L + j
                s = None
                for q in range(D // L):
                    eh = hrows[row, pl.ds(q * L, L)]
                    er = rrows[row, pl.ds(q * L, L)]
                    et = trows[row, pl.ds(q * L, L)]
                    d = (eh - et) + er
                    s = d * d if s is None else s + d * d
                tot = jnp.sum(s)
                sums = jnp.where(lanes == j, tot, sums)
            x = sums + 1e-12
            i = plsc.bitcast(x, jnp.int32)
            i = 0x5F3759DF - lax.shift_right_logical(i, 1)
            y = plsc.bitcast(i, jnp.float32)
            for _ in range(3):
                y = y * (1.5 - 0.5 * x * y * y)
            outv[pl.ds(c * CHUNK + g * L, L)] = -(x * y)
            return carry

        lax.fori_loop(0, GROUPS, group, 0)

    pltpu.sync_copy(outv, out.at[pl.ds(wid * BPW, BPW)])


@jax.jit
def kernel(h, t, r, entity_emb, relation_emb):
    h2 = h.astype(jnp.int32).reshape(B // CHUNK, CHUNK)
    t2 = t.astype(jnp.int32).reshape(B // CHUNK, CHUNK)
    r2 = r.astype(jnp.int32).reshape(B // CHUNK, CHUNK)
    mesh = plsc.VectorSubcoreMesh(
        core_axis_name="c", subcore_axis_name="s",
        num_cores=NC, num_subcores=NS)
    run = pl.kernel(
        _score_body,
        out_type=jax.ShapeDtypeStruct((B,), jnp.float32),
        mesh=mesh,
        compiler_params=pltpu.CompilerParams(
            needs_layout_passes=False, use_tc_tiling_on_sc=False),
        scratch_types=[
            pltpu.VMEM((NCHUNK, CHUNK), jnp.int32),
            pltpu.VMEM((NCHUNK, CHUNK), jnp.int32),
            pltpu.VMEM((NCHUNK, CHUNK), jnp.int32),
            pltpu.VMEM((CHUNK, D), jnp.float32),
            pltpu.VMEM((CHUNK, D), jnp.float32),
            pltpu.VMEM((CHUNK, D), jnp.float32),
            pltpu.VMEM((BPW,), jnp.float32),
            pltpu.SemaphoreType.DMA,
            pltpu.SemaphoreType.DMA,
            pltpu.SemaphoreType.DMA,
        ],
    )
    return run(h2, t2, r2, entity_emb, entity_emb, relation_emb)
